# Initial kernel scaffold; baseline (speedup 1.0000x reference)
import jax
import jax.numpy as jnp
from jax.experimental import pallas as pl


def kernel(x, edge_weight, batch, params):
    raise NotImplementedError("write your pallas kernel here")


# speedup vs baseline: 3.1409x; 3.1409x over previous
import functools

import jax
import jax.numpy as jnp
from jax import lax
from jax.experimental import pallas as pl
from jax.experimental.pallas import tpu as pltpu
from jax.experimental.pallas import tpu_sc as plsc

N = 10000
IN = 128
HID = 256
M = 64
OUT = 64

_R = 256
_C = 512
_RB = 1000
_INF = float('inf')

_SELU_SCALE = 1.0507009873554804934193349852946
_SELU_ALPHA = 1.6732632423543772848170429916717


def _selu(x):
    return _SELU_SCALE * jnp.where(x > 0, x, _SELU_ALPHA * (jnp.exp(x) - 1.0))


def _knn3_body(xr_ref, xt_ref, sqr_ref, sqc_ref, out_ref):
    i = pl.program_id(0)
    xr = xr_ref[...]
    sqr = sqr_ref[...]
    grow = i * _R + lax.broadcasted_iota(jnp.int32, (_R, 1), 0)
    npad = xt_ref.shape[1]

    def chunk_step(c, carry):
        b0, b1, b2, i0, i1, i2 = carry
        xc = xt_ref[:, pl.ds(c * _C, _C)]
        sqc = sqc_ref[:, pl.ds(c * _C, _C)]
        dot = jnp.dot(xr, xc, preferred_element_type=jnp.float32)
        s = (sqr - 2.0 * dot) + sqc
        gc = c * _C + lax.broadcasted_iota(jnp.int32, (1, _C), 1)
        s = jnp.where((grow == gc) | (gc >= N), _INF, s)

        def insert(v, ix, b0, b1, b2, i0, i1, i2):
            lt0, lt1, lt2 = v < b0, v < b1, v < b2
            nb0 = jnp.where(lt0, v, b0)
            ni0 = jnp.where(lt0, ix, i0)
            nb1 = jnp.where(lt0, b0, jnp.where(lt1, v, b1))
            ni1 = jnp.where(lt0, i0, jnp.where(lt1, ix, i1))
            nb2 = jnp.where(lt1, b1, jnp.where(lt2, v, b2))
            ni2 = jnp.where(lt1, i1, jnp.where(lt2, ix, i2))
            return nb0, nb1, nb2, ni0, ni1, ni2

        for t in range(3):
            m = jnp.min(s, axis=1, keepdims=True)
            ix = jnp.min(jnp.where(s == m, gc, N), axis=1, keepdims=True)
            b0, b1, b2, i0, i1, i2 = insert(m, ix, b0, b1, b2, i0, i1, i2)
            if t < 2:
                s = jnp.where(gc == ix, _INF, s)
        return b0, b1, b2, i0, i1, i2

    finf = jnp.full((_R, 1), _INF, jnp.float32)
    zi = jnp.zeros((_R, 1), jnp.int32)
    carry = (finf, finf, finf, zi, zi, zi)
    _, _, _, i0, i1, i2 = lax.fori_loop(0, npad // _C, chunk_step, carry)
    out_ref[...] = jnp.concatenate([i0, i1, i2, i0, i0, i0, i0, i0], axis=1)


def _knn3(x):
    n, f = x.shape
    npad = ((n + _C - 1) // _C) * _C
    xt = jnp.pad(x.T, ((0, 0), (0, npad - n)))
    sq = jnp.sum(x * x, axis=1)
    out = pl.pallas_call(
        _knn3_body,
        grid=((n + _R - 1) // _R,),
        in_specs=[
            pl.BlockSpec((_R, f), lambda i: (i, 0)),
            pl.BlockSpec((f, npad), lambda i: (0, 0)),
            pl.BlockSpec((_R, 1), lambda i: (i, 0)),
            pl.BlockSpec((1, npad), lambda i: (0, 0)),
        ],
        out_specs=pl.BlockSpec((_R, 8), lambda i: (i, 0)),
        out_shape=jax.ShapeDtypeStruct((n, 8), jnp.int32),
    )(x, xt, sq[:, None], jnp.pad(sq[None, :], ((0, 0), (0, npad - n))))
    return out[:, :3]


_B_PAD = 30720
_CHUNK = 240


def _sc_gather(table, idx):
    V, D = table.shape
    B = idx.shape[0]
    nw = 32
    b_per_w = B // nw
    nch = b_per_w // _CHUNK
    mesh = plsc.VectorSubcoreMesh(core_axis_name="c", subcore_axis_name="s")

    @functools.partial(
        pl.kernel, mesh=mesh,
        out_type=jax.ShapeDtypeStruct((B, D), jnp.float32),
        scratch_types=[
            pltpu.VMEM((b_per_w,), jnp.int32),
            pltpu.VMEM((_CHUNK, D), jnp.float32),
            pltpu.SemaphoreType.DMA,
        ],
    )
    def k(table_hbm, idx_hbm, out_hbm, idx_v, rows_v, sem):
        wid = lax.axis_index("s") * 2 + lax.axis_index("c")
        base = wid * b_per_w
        pltpu.sync_copy(idx_hbm.at[pl.ds(base, b_per_w)], idx_v)

        def step(c, _):
            pltpu.async_copy(
                table_hbm.at[idx_v.at[pl.ds(c * _CHUNK, _CHUNK)]],
                rows_v, sem).wait()
            pltpu.sync_copy(rows_v, out_hbm.at[pl.ds(base + c * _CHUNK, _CHUNK)])
            return 0

        lax.fori_loop(0, nch, step, 0, unroll=False)

    return k(table, idx)


def _deg_body(w_ref, out_ref):
    deg = (w_ref[:, 0:1] + w_ref[:, 1:2]) + w_ref[:, 2:3]
    dinv = 1.0 / jnp.sqrt(deg + 1e-12)
    out_ref[...] = jnp.broadcast_to(dinv, out_ref.shape)


def _k_deg(w8):
    return pl.pallas_call(
        _deg_body,
        grid=(N // _RB,),
        in_specs=[pl.BlockSpec((_RB, 8), lambda i: (i, 0))],
        out_specs=pl.BlockSpec((_RB, 8), lambda i: (i, 0)),
        out_shape=jax.ShapeDtypeStruct((N, 8), jnp.float32),
    )(w8)


def _prop_body(rows_ref, din_ref, w_ref, dv_ref, out_ref, *, F):
    p = None
    dvs = dv_ref[:, 0:1]
    for j in range(3):
        wj = w_ref[:, j:j + 1]
        dsj = din_ref[:, 128 * j:128 * j + 1]
        nj = (wj * dsj) * dvs
        aj = nj * rows_ref[:, F * j:F * (j + 1)]
        p = aj if p is None else p + aj
    out_ref[...] = p


def _k_prop(rows, din, w8, dinv8, F):
    return pl.pallas_call(
        functools.partial(_prop_body, F=F),
        grid=(N // _RB,),
        in_specs=[
            pl.BlockSpec((_RB, 3 * F), lambda i: (i, 0)),
            pl.BlockSpec((_RB, 384), lambda i: (i, 0)),
            pl.BlockSpec((_RB, 8), lambda i: (i, 0)),
            pl.BlockSpec((_RB, 8), lambda i: (i, 0)),
        ],
        out_specs=pl.BlockSpec((_RB, F), lambda i: (i, 0)),
        out_shape=jax.ShapeDtypeStruct((N, F), jnp.float32),
    )(rows, din, w8, dinv8)


def _cheb_body(x_ref, p1_ref, p2_ref, w_ref, b_ref, out_ref, *, F):
    x = x_ref[...]
    t2 = 2.0 * p2_ref[...] - x
    out = jnp.dot(x, w_ref[0:F], preferred_element_type=jnp.float32)
    out = out + jnp.dot(-p1_ref[...], w_ref[F:2 * F],
                        preferred_element_type=jnp.float32)
    out = out + jnp.dot(t2, w_ref[2 * F:3 * F],
                        preferred_element_type=jnp.float32)
    out_ref[...] = out + b_ref[...]


def _k_cheb(x, p1, p2, W2d, b, F, G):
    return pl.pallas_call(
        functools.partial(_cheb_body, F=F),
        grid=(N // _RB,),
        in_specs=[
            pl.BlockSpec((_RB, F), lambda i: (i, 0)),
            pl.BlockSpec((_RB, F), lambda i: (i, 0)),
            pl.BlockSpec((_RB, F), lambda i: (i, 0)),
            pl.BlockSpec((3 * F, G), lambda i: (0, 0)),
            pl.BlockSpec((1, G), lambda i: (0, 0)),
        ],
        out_specs=pl.BlockSpec((_RB, G), lambda i: (i, 0)),
        out_shape=jax.ShapeDtypeStruct((N, G), jnp.float32),
    )(x, p1, p2, W2d, b)


def _cheb_layer(t, nbr, w8, dinv8, dinv128, W, b):
    F = t.shape[1]
    G = W.shape[2]
    src = jnp.pad(nbr.reshape(-1), (0, _B_PAD - 3 * N))
    gdin = _sc_gather(dinv128, src)[:3 * N].reshape(N, 384)
    gt = _sc_gather(t, src)[:3 * N].reshape(N, 3 * F)
    p1 = _k_prop(gt, gdin, w8, dinv8, F)
    gp1 = _sc_gather(p1, src)[:3 * N].reshape(N, 3 * F)
    p2 = _k_prop(gp1, gdin, w8, dinv8, F)
    return _k_cheb(t, p1, p2, W.reshape(3 * F, G), b[None, :], F, G)


def _gsum_body(a_ref, out_ref, acc):
    @pl.when(pl.program_id(0) == 0)
    def _():
        acc[...] = jnp.zeros_like(acc)
    acc[...] += jnp.sum(a_ref[...], axis=0, keepdims=True)
    @pl.when(pl.program_id(0) == pl.num_programs(0) - 1)
    def _():
        out_ref[...] = acc[...] * (1.0 / N)


def _k_mean(u):
    G = u.shape[1]
    return pl.pallas_call(
        _gsum_body,
        grid=(N // _RB,),
        in_specs=[pl.BlockSpec((_RB, G), lambda i: (i, 0))],
        out_specs=pl.BlockSpec((1, G), lambda i: (0, 0)),
        out_shape=jax.ShapeDtypeStruct((1, G), jnp.float32),
        scratch_shapes=[pltpu.VMEM((1, G), jnp.float32)],
    )(u)


def _gvar_body(a_ref, m_ref, al_ref, out_ref, acc):
    @pl.when(pl.program_id(0) == 0)
    def _():
        acc[...] = jnp.zeros_like(acc)
    uc = a_ref[...] - al_ref[...] * m_ref[...]
    acc[...] += jnp.sum(uc * uc, axis=0, keepdims=True)
    @pl.when(pl.program_id(0) == pl.num_programs(0) - 1)
    def _():
        out_ref[...] = acc[...] * (1.0 / N)


def _k_var(u, mean, alpha):
    G = u.shape[1]
    return pl.pallas_call(
        _gvar_body,
        grid=(N // _RB,),
        in_specs=[pl.BlockSpec((_RB, G), lambda i: (i, 0)),
                  pl.BlockSpec((1, G), lambda i: (0, 0)),
                  pl.BlockSpec((1, G), lambda i: (0, 0))],
        out_specs=pl.BlockSpec((1, G), lambda i: (0, 0)),
        out_shape=jax.ShapeDtypeStruct((1, G), jnp.float32),
        scratch_shapes=[pltpu.VMEM((1, G), jnp.float32)],
    )(u, mean, alpha[None, :])


def _gnorm_body(a_ref, m_ref, v_ref, al_ref, g_ref, be_ref, out_ref):
    uc = a_ref[...] - al_ref[...] * m_ref[...]
    out_ref[...] = g_ref[...] * uc / jnp.sqrt(v_ref[...] + 1e-5) + be_ref[...]


def _k_gnorm(u, mean, var, alpha, gamma, beta):
    G = u.shape[1]
    one = lambda a: a[None, :]
    return pl.pallas_call(
        _gnorm_body,
        grid=(N // _RB,),
        in_specs=[pl.BlockSpec((_RB, G), lambda i: (i, 0))] +
                 [pl.BlockSpec((1, G), lambda i: (0, 0))] * 5,
        out_specs=pl.BlockSpec((_RB, G), lambda i: (i, 0)),
        out_shape=jax.ShapeDtypeStruct((N, G), jnp.float32),
    )(u, mean, var, one(alpha), one(gamma), one(beta))


def _fc1_body(u_ref, w1, b1, w2, b2, w3, b3, out_ref):
    h = _selu(jnp.dot(u_ref[...], w1[...], preferred_element_type=jnp.float32)
              + b1[...])
    h = _selu(jnp.dot(h, w2[...], preferred_element_type=jnp.float32) + b2[...])
    out_ref[...] = jnp.dot(h, w3[...], preferred_element_type=jnp.float32) + b3[...]


def _k_fc1(u, p):
    one = lambda a: a[None, :]
    ws = [p['fc1_w1'], one(p['fc1_b1']), p['fc1_w2'], one(p['fc1_b2']),
          p['fc1_w3'], one(p['fc1_b3'])]
    specs = [pl.BlockSpec(w.shape, lambda i: (0, 0)) for w in ws]
    return pl.pallas_call(
        _fc1_body,
        grid=(N // _RB,),
        in_specs=[pl.BlockSpec((_RB, 4 * M), lambda i: (i, 0))] + specs,
        out_specs=pl.BlockSpec((_RB, M), lambda i: (i, 0)),
        out_shape=jax.ShapeDtypeStruct((N, M), jnp.float32),
    )(u, *ws)


def _pool_body(l_ref, x_ref, xp_ref, cs_ref, axp, acs):
    @pl.when(pl.program_id(0) == 0)
    def _():
        axp[...] = jnp.zeros_like(axp)
        acs[...] = jnp.zeros_like(acs)
    z = l_ref[...]
    z = z - jnp.max(z, axis=1, keepdims=True)
    e = jnp.exp(z)
    S = e / jnp.sum(e, axis=1, keepdims=True)
    axp[...] += lax.dot_general(S, x_ref[...], (((0,), (0,)), ((), ())),
                                preferred_element_type=jnp.float32)
    acs[...] += lax.dot_general(S, jnp.ones((S.shape[0], 8), jnp.float32),
                                (((0,), (0,)), ((), ())),
                                precision=lax.Precision.HIGHEST,
                                preferred_element_type=jnp.float32)
    @pl.when(pl.program_id(0) == pl.num_programs(0) - 1)
    def _():
        xp_ref[...] = axp[...]
        cs_ref[...] = acs[...]


def _k_pool(logits, x):
    return pl.pallas_call(
        _pool_body,
        grid=(N // _RB,),
        in_specs=[pl.BlockSpec((_RB, M), lambda i: (i, 0)),
                  pl.BlockSpec((_RB, IN), lambda i: (i, 0))],
        out_specs=[pl.BlockSpec((M, IN), lambda i: (0, 0)),
                   pl.BlockSpec((M, 8), lambda i: (0, 0))],
        out_shape=[jax.ShapeDtypeStruct((M, IN), jnp.float32),
                   jax.ShapeDtypeStruct((M, 8), jnp.float32)],
        scratch_shapes=[pltpu.VMEM((M, IN), jnp.float32),
                        pltpu.VMEM((M, 8), jnp.float32)],
    )(logits, x)


def _topk_small(d, k):
    gc = lax.broadcasted_iota(jnp.int32, (1, M), 1)
    idxs = []
    for t in range(k):
        m = jnp.min(d, axis=1, keepdims=True)
        ix = jnp.min(jnp.where(d == m, gc, M), axis=1, keepdims=True)
        idxs.append(ix)
        if t < k - 1:
            d = jnp.where(gc == ix, _INF, d)
    return idxs


def _onehot_gather(t, ix):
    gc = lax.broadcasted_iota(jnp.int32, (1, M), 1)
    oh = (ix == gc).astype(jnp.float32)
    return lax.dot_general(oh, t, (((1,), (0,)), ((), ())),
                           precision=lax.Precision.HIGHEST,
                           preferred_element_type=jnp.float32)


def _sq_cols(t):
    ts = t * t
    return lax.dot_general(jnp.ones((1, t.shape[1]), jnp.float32), ts,
                           (((1,), (1,)), ((), ())),
                           precision=lax.Precision.HIGHEST,
                           preferred_element_type=jnp.float32)


def _edgeconv_small(t, k, w_ref, b_ref):
    sqr = jnp.sum(t * t, axis=1, keepdims=True)
    dotm = lax.dot_general(t, t, (((1,), (1,)), ((), ())),
                           preferred_element_type=jnp.float32)
    d = (sqr - 2.0 * dotm) + _sq_cols(t)
    diag = (lax.broadcasted_iota(jnp.int32, (M, 1), 0) ==
            lax.broadcasted_iota(jnp.int32, (1, M), 1))
    d = jnp.where(diag, _INF, d)
    idxs = _topk_small(d, k)
    h = None
    for ix in idxs:
        ts = _onehot_gather(t, ix)
        cat = jnp.concatenate([t, ts - t], axis=1)
        msg = _selu(jnp.dot(cat, w_ref[...], preferred_element_type=jnp.float32)
                    + b_ref[...])
        h = msg if h is None else jnp.maximum(h, msg)
    return h


def _tail_body(xpn_ref, cs_ref, w2, b2, w3, b3, f1w, f1b, f2w, f2b, f3w, f3b,
               out_ref):
    xp = xpn_ref[...] / (cs_ref[:, 0:1] + 1e-12)
    h = _edgeconv_small(xp, 8, w2, b2)
    y = _edgeconv_small(h, 3, w3, b3)
    z = h + y
    v = _selu(jnp.dot(z, f1w[...], preferred_element_type=jnp.float32) + f1b[...])
    v = _selu(jnp.dot(v, f2w[...], preferred_element_type=jnp.float32) + f2b[...])
    out_ref[...] = jnp.dot(v, f3w[...], preferred_element_type=jnp.float32) + f3b[...]


def _k_tail(xpn, cs, p):
    one = lambda a: a[None, :]
    ws = [p['conv2_w'], one(p['conv2_b']), p['conv3_w'], one(p['conv3_b']),
          p['fc2_w1'], one(p['fc2_b1']), p['fc2_w2'], one(p['fc2_b2']),
          p['fc2_w3'], one(p['fc2_b3'])]
    specs = [pl.BlockSpec(w.shape, lambda i: (0, 0)) for w in ws]
    return pl.pallas_call(
        _tail_body,
        grid=(1,),
        in_specs=[pl.BlockSpec((M, IN), lambda i: (0, 0)),
                  pl.BlockSpec((M, 8), lambda i: (0, 0))] + specs,
        out_specs=pl.BlockSpec((M, OUT), lambda i: (0, 0)),
        out_shape=jax.ShapeDtypeStruct((M, OUT), jnp.float32),
    )(xpn, cs, *ws)


def kernel(x, edge_weight, batch, params):
    p = params
    w8 = jnp.pad(edge_weight.reshape(N, 3), ((0, 0), (0, 5)))
    dinv8 = _k_deg(w8)
    dinv128 = jnp.broadcast_to(dinv8[:, 0:1], (N, 128))

    nbr1 = _knn3(x)
    u = _cheb_layer(x, nbr1, w8, dinv8, dinv128, p['conv0_W'], p['conv0_b'])

    mean = _k_mean(u)
    var = _k_var(u, mean, p['gn_alpha'])
    u = _k_gnorm(u, mean, var, p['gn_alpha'], p['gn_gamma'], p['gn_beta'])

    nbr2 = _knn3(u)
    u = _cheb_layer(u, nbr2, w8, dinv8, dinv128, p['conv1_W'], p['conv1_b'])

    logits = _k_fc1(u, p)
    xpn, cs = _k_pool(logits, x)
    v = _k_tail(xpn, cs, p)
    return (logits, v)
